# Initial kernel scaffold; baseline (speedup 1.0000x reference)
import jax
import jax.numpy as jnp
from jax.experimental import pallas as pl


def kernel(centerKpoints, target_wh, output_hm, output_wh, output_STA_offset, mask, index):
    raise NotImplementedError("write your pallas kernel here")


# speedup vs baseline: 59.7578x; 59.7578x over previous
import jax
import jax.numpy as jnp
from jax import lax
from jax.experimental import pallas as pl
from jax.experimental.pallas import tpu as pltpu

_B, _NOBJ, _K, _C, _H, _W = 32, 32, 7, 24, 192, 192
_HW = _H * _W
_CHW = _C * _HW
_SUB = _CHW // 128
_K2 = 2 * _K
_OFFSET_W_RATIO = 1.0
_OFFSET_H_RATIO = 1.0
_HH = 1.0
_EPS = 1e-07


def _argmax_body(hm_ref, out_ref):
    v = hm_ref[0]
    m = jnp.max(v)
    row = lax.broadcasted_iota(jnp.int32, (_SUB, 128), 0)
    col = lax.broadcasted_iota(jnp.int32, (_SUB, 128), 1)
    flat = row * 128 + col
    fm = jnp.min(jnp.where(v == m, flat, jnp.int32(_CHW)))
    p = fm % _HW
    out_ref[0] = jnp.full((1, 8), p, jnp.int32)


def _loss_body(idx0_ref, p_ref, wh_ref, off_ref, ckp_ref, tgt_ref, msk_ref,
               sin_ref, cos_ref):
    b = pl.program_id(0)
    i0 = idx0_ref[b]
    pp = p_ref[b]
    r1 = i0 % 128
    r2 = pp % 128
    lane = lax.broadcasted_iota(jnp.int32, (_K2, 128), 1)
    pred = jnp.sum(jnp.where(lane == r1, wh_ref[0], 0.0), axis=1, keepdims=True)
    off = jnp.sum(jnp.where(lane == r2, off_ref[0], 0.0), axis=1, keepdims=True)
    msk = msk_ref[0]
    ckp = ckp_ref[0] * msk
    tgt = tgt_ref[0] * msk
    xs = (pp % _W).astype(jnp.float32)
    ys = (pp // _W).astype(jnp.float32)
    comp = lax.broadcasted_iota(jnp.int32, (_K2, 1), 0) % 2
    iseven = comp == 0
    pos = jnp.where(iseven, xs, ys)
    offr = off * jnp.where(iseven, _OFFSET_W_RATIO, _OFFSET_H_RATIO)
    p1 = pos + offr - pred * 0.5
    p2 = pos + offr + pred * 0.5
    g1 = ckp - tgt * 0.5
    g2 = ckp + tgt * 0.5
    pc = (p1 + p2) / 2
    gc = (g1 + g2) / 2

    z2 = jnp.zeros((2, 1), jnp.float32)
    z1 = jnp.zeros((1, 1), jnp.float32)

    def sh2(v):
        return jnp.concatenate([v[2:], z2], axis=0)

    def pair(v):
        return v + jnp.concatenate([v[1:], z1], axis=0)

    vgg = sh2(gc) - gc
    vpp = sh2(pc) - pc
    vgp = sh2(gc) - pc
    vpg = sh2(pc) - gc
    dp = pc - gc
    dn = sh2(pc) - sh2(gc)

    d2p = pair(dp * dp) + _EPS
    d2n = pair(dn * dn) + _EPS
    n2p = pair(vgp * vgp) + _HH
    n2n = pair(vpg * vpg) + _HH
    sin = (jnp.sqrt(d2p) / jnp.sqrt(n2p) + jnp.sqrt(d2n) / jnp.sqrt(n2n)) / 2
    cross = (pair(vgp * vpg) + _HH) / (jnp.sqrt(n2p) * jnp.sqrt(n2n))
    own = (pair(vgg * vpp) + _HH) / (
        jnp.sqrt(pair(vgg * vgg) + _HH) * jnp.sqrt(pair(vpp * vpp) + _HH))
    cos = 1.0 - (cross + own) / 2
    sin_ref[0] = jnp.concatenate([0.5 * sin, z2], axis=0)
    cos_ref[0] = jnp.concatenate([0.5 * cos, z2], axis=0)


def kernel(centerKpoints, target_wh, output_hm, output_wh, output_STA_offset,
           mask, index):
    hm3 = output_hm.reshape(_B, _SUB, 128)
    p3 = pl.pallas_call(
        _argmax_body,
        grid=(_B,),
        in_specs=[pl.BlockSpec((1, _SUB, 128), lambda b: (b, 0, 0))],
        out_specs=pl.BlockSpec((1, 1, 8), lambda b: (b, 0, 0)),
        out_shape=jax.ShapeDtypeStruct((_B, 1, 8), jnp.int32),
    )(hm3)
    p = p3[:, 0, 0]
    idx0 = index[:, 0].astype(jnp.int32)

    wh = output_wh.reshape(_B, _K2, _HW)
    off = output_STA_offset.reshape(_B, _K2, _HW)
    ckp = centerKpoints[:, 0, :].reshape(_B, _K2, 1)
    tgt = target_wh[:, 0, :].reshape(_B, _K2, 1)
    msk = jnp.broadcast_to(mask[:, 0].reshape(_B, 1, 1), (_B, _K2, 1))

    sin3, cos3 = pl.pallas_call(
        _loss_body,
        grid_spec=pltpu.PrefetchScalarGridSpec(
            num_scalar_prefetch=2,
            grid=(_B,),
            in_specs=[
                pl.BlockSpec((1, _K2, 128), lambda b, i0, p_: (b, 0, i0[b] // 128)),
                pl.BlockSpec((1, _K2, 128), lambda b, i0, p_: (b, 0, p_[b] // 128)),
                pl.BlockSpec((1, _K2, 1), lambda b, i0, p_: (b, 0, 0)),
                pl.BlockSpec((1, _K2, 1), lambda b, i0, p_: (b, 0, 0)),
                pl.BlockSpec((1, _K2, 1), lambda b, i0, p_: (b, 0, 0)),
            ],
            out_specs=[
                pl.BlockSpec((1, 16, 1), lambda b, i0, p_: (b, 0, 0)),
                pl.BlockSpec((1, 16, 1), lambda b, i0, p_: (b, 0, 0)),
            ],
        ),
        out_shape=[jax.ShapeDtypeStruct((_B, 16, 1), jnp.float32)] * 2,
    )(idx0, p, wh, off, ckp, tgt, msk)

    sin = sin3[:, 0:12:2, 0].reshape(-1)
    cos = cos3[:, 0:12:2, 0].reshape(-1)
    return sin, cos


# speedup vs baseline: 206.8056x; 3.4607x over previous
import jax
import jax.numpy as jnp
from jax import lax
from jax.experimental import pallas as pl
from jax.experimental.pallas import tpu as pltpu

_B, _NOBJ, _K, _C, _H, _W = 32, 32, 7, 24, 192, 192
_HW = _H * _W
_CHW = _C * _HW
_SUB = _CHW // 128
_K2 = 2 * _K
_OFFSET_W_RATIO = 1.0
_OFFSET_H_RATIO = 1.0
_HH = 1.0
_EPS = 1e-07


def _argmax_body(hm_ref, out_ref):
    v = hm_ref[0].reshape(_C * _H, _W)
    m = jnp.max(v)
    row = lax.broadcasted_iota(jnp.int32, (_C * _H, _W), 0)
    col = lax.broadcasted_iota(jnp.int32, (_C * _H, _W), 1)
    flat = row * _W + col
    fm = jnp.min(jnp.where(v == m, flat, jnp.int32(_CHW)))
    p = fm % _HW
    out_ref[0] = jnp.full((1, 8), p, jnp.int32)


def _loss_body(idx0_ref, p_ref, wh_ref, off_ref, ckp_ref, tgt_ref, msk_ref,
               sin_ref, cos_ref):
    b = pl.program_id(0)
    i0 = idx0_ref[b]
    pp = p_ref[b]

    def _extract(blk, y, x):
        rowm = lax.broadcasted_iota(jnp.int32, (_K2, 8, 128), 1) == (y % 8)
        lanem = lax.broadcasted_iota(jnp.int32, (_K2, 8, 128), 2) == (x % 128)
        t = jnp.where(rowm & lanem, blk, 0.0)
        return jnp.sum(jnp.sum(t, axis=2), axis=1, keepdims=True)

    pred = _extract(wh_ref[0], i0 // _W, i0 % _W)
    off = _extract(off_ref[0], pp // _W, pp % _W)
    msk = msk_ref[0]
    ckp = ckp_ref[0] * msk
    tgt = tgt_ref[0] * msk
    xs = (pp % _W).astype(jnp.float32)
    ys = (pp // _W).astype(jnp.float32)
    comp = lax.broadcasted_iota(jnp.int32, (_K2, 1), 0) % 2
    iseven = comp == 0
    pos = jnp.where(iseven, xs, ys)
    offr = off * jnp.where(iseven, _OFFSET_W_RATIO, _OFFSET_H_RATIO)
    p1 = pos + offr - pred * 0.5
    p2 = pos + offr + pred * 0.5
    g1 = ckp - tgt * 0.5
    g2 = ckp + tgt * 0.5
    pc = (p1 + p2) / 2
    gc = (g1 + g2) / 2

    z2 = jnp.zeros((2, 1), jnp.float32)
    z1 = jnp.zeros((1, 1), jnp.float32)

    def sh2(v):
        return jnp.concatenate([v[2:], z2], axis=0)

    def pair(v):
        return v + jnp.concatenate([v[1:], z1], axis=0)

    vgg = sh2(gc) - gc
    vpp = sh2(pc) - pc
    vgp = sh2(gc) - pc
    vpg = sh2(pc) - gc
    dp = pc - gc
    dn = sh2(pc) - sh2(gc)

    d2p = pair(dp * dp) + _EPS
    d2n = pair(dn * dn) + _EPS
    n2p = pair(vgp * vgp) + _HH
    n2n = pair(vpg * vpg) + _HH
    sin = (jnp.sqrt(d2p) / jnp.sqrt(n2p) + jnp.sqrt(d2n) / jnp.sqrt(n2n)) / 2
    cross = (pair(vgp * vpg) + _HH) / (jnp.sqrt(n2p) * jnp.sqrt(n2n))
    own = (pair(vgg * vpp) + _HH) / (
        jnp.sqrt(pair(vgg * vgg) + _HH) * jnp.sqrt(pair(vpp * vpp) + _HH))
    cos = 1.0 - (cross + own) / 2
    sin_ref[0] = jnp.concatenate([0.5 * sin, z2], axis=0)
    cos_ref[0] = jnp.concatenate([0.5 * cos, z2], axis=0)


def kernel(centerKpoints, target_wh, output_hm, output_wh, output_STA_offset,
           mask, index):
    p3 = pl.pallas_call(
        _argmax_body,
        grid=(_B,),
        in_specs=[pl.BlockSpec((1, _C, _H, _W), lambda b: (b, 0, 0, 0))],
        out_specs=pl.BlockSpec((1, 1, 8), lambda b: (b, 0, 0)),
        out_shape=jax.ShapeDtypeStruct((_B, 1, 8), jnp.int32),
    )(output_hm)
    p = p3[:, 0, 0]
    idx0 = index[:, 0].astype(jnp.int32)
    ckp = centerKpoints[:, 0, :].reshape(_B, _K2, 1)
    tgt = target_wh[:, 0, :].reshape(_B, _K2, 1)
    msk = jnp.broadcast_to(mask[:, 0].reshape(_B, 1, 1), (_B, _K2, 1))

    sin3, cos3 = pl.pallas_call(
        _loss_body,
        grid_spec=pltpu.PrefetchScalarGridSpec(
            num_scalar_prefetch=2,
            grid=(_B,),
            in_specs=[
                pl.BlockSpec((1, _K2, 8, 128),
                             lambda b, i0, p_: (b, 0, (i0[b] // _W) // 8,
                                                (i0[b] % _W) // 128)),
                pl.BlockSpec((1, _K2, 8, 128),
                             lambda b, i0, p_: (b, 0, (p_[b] // _W) // 8,
                                                (p_[b] % _W) // 128)),
                pl.BlockSpec((1, _K2, 1), lambda b, i0, p_: (b, 0, 0)),
                pl.BlockSpec((1, _K2, 1), lambda b, i0, p_: (b, 0, 0)),
                pl.BlockSpec((1, _K2, 1), lambda b, i0, p_: (b, 0, 0)),
            ],
            out_specs=[
                pl.BlockSpec((1, 16, 1), lambda b, i0, p_: (b, 0, 0)),
                pl.BlockSpec((1, 16, 1), lambda b, i0, p_: (b, 0, 0)),
            ],
        ),
        out_shape=[jax.ShapeDtypeStruct((_B, 16, 1), jnp.float32)] * 2,
    )(idx0, p, output_wh, output_STA_offset, ckp, tgt, msk)

    sin = sin3[:, 0:12:2, 0].reshape(-1)
    cos = cos3[:, 0:12:2, 0].reshape(-1)
    return sin, cos


# speedup vs baseline: 253.7435x; 1.2270x over previous
import jax
import jax.numpy as jnp
from jax import lax
from jax.experimental import pallas as pl
from jax.experimental.pallas import tpu as pltpu

_B, _NOBJ, _K, _C, _H, _W = 32, 32, 7, 24, 192, 192
_HW = _H * _W
_CHW = _C * _HW
_SUB = _CHW // 128
_K2 = 2 * _K
_OFFSET_W_RATIO = 1.0
_OFFSET_H_RATIO = 1.0
_HH = 1.0
_EPS = 1e-07


def _argmax_body(hm_ref, out_ref):
    vc = hm_ref[0]
    cm = jnp.max(jnp.max(vc, axis=1), axis=1)
    m = jnp.max(cm)
    c_iota = lax.broadcasted_iota(jnp.int32, (_C,), 0)
    cstar = jnp.min(jnp.where(cm == m, c_iota, jnp.int32(_C)))
    blk = hm_ref[0, cstar]
    row = lax.broadcasted_iota(jnp.int32, (_H, _W), 0)
    col = lax.broadcasted_iota(jnp.int32, (_H, _W), 1)
    flat = row * _W + col
    p = jnp.min(jnp.where(blk == m, flat, jnp.int32(_HW)))
    out_ref[0] = jnp.full((1, 8), p, jnp.int32)


def _loss_body(idx0_ref, p_ref, wh_ref, off_ref, ckp_ref, tgt_ref, msk_ref,
               sin_ref, cos_ref):
    b = pl.program_id(0)
    i0 = idx0_ref[b]
    pp = p_ref[b]

    def _extract(blk, y, x):
        rowm = lax.broadcasted_iota(jnp.int32, (_K2, 8, 128), 1) == (y % 8)
        lanem = lax.broadcasted_iota(jnp.int32, (_K2, 8, 128), 2) == (x % 128)
        t = jnp.where(rowm & lanem, blk, 0.0)
        return jnp.sum(jnp.sum(t, axis=2), axis=1, keepdims=True)

    pred = _extract(wh_ref[0], i0 // _W, i0 % _W)
    off = _extract(off_ref[0], pp // _W, pp % _W)
    msk = msk_ref[0]
    ckp = ckp_ref[0] * msk
    tgt = tgt_ref[0] * msk
    xs = (pp % _W).astype(jnp.float32)
    ys = (pp // _W).astype(jnp.float32)
    comp = lax.broadcasted_iota(jnp.int32, (_K2, 1), 0) % 2
    iseven = comp == 0
    pos = jnp.where(iseven, xs, ys)
    offr = off * jnp.where(iseven, _OFFSET_W_RATIO, _OFFSET_H_RATIO)
    p1 = pos + offr - pred * 0.5
    p2 = pos + offr + pred * 0.5
    g1 = ckp - tgt * 0.5
    g2 = ckp + tgt * 0.5
    pc = (p1 + p2) / 2
    gc = (g1 + g2) / 2

    z2 = jnp.zeros((2, 1), jnp.float32)
    z1 = jnp.zeros((1, 1), jnp.float32)

    def sh2(v):
        return jnp.concatenate([v[2:], z2], axis=0)

    def pair(v):
        return v + jnp.concatenate([v[1:], z1], axis=0)

    vgg = sh2(gc) - gc
    vpp = sh2(pc) - pc
    vgp = sh2(gc) - pc
    vpg = sh2(pc) - gc
    dp = pc - gc
    dn = sh2(pc) - sh2(gc)

    d2p = pair(dp * dp) + _EPS
    d2n = pair(dn * dn) + _EPS
    n2p = pair(vgp * vgp) + _HH
    n2n = pair(vpg * vpg) + _HH
    sin = (jnp.sqrt(d2p) / jnp.sqrt(n2p) + jnp.sqrt(d2n) / jnp.sqrt(n2n)) / 2
    cross = (pair(vgp * vpg) + _HH) / (jnp.sqrt(n2p) * jnp.sqrt(n2n))
    own = (pair(vgg * vpp) + _HH) / (
        jnp.sqrt(pair(vgg * vgg) + _HH) * jnp.sqrt(pair(vpp * vpp) + _HH))
    cos = 1.0 - (cross + own) / 2
    sin_ref[0] = jnp.concatenate([0.5 * sin, z2], axis=0)
    cos_ref[0] = jnp.concatenate([0.5 * cos, z2], axis=0)


def kernel(centerKpoints, target_wh, output_hm, output_wh, output_STA_offset,
           mask, index):
    p3 = pl.pallas_call(
        _argmax_body,
        grid=(_B,),
        in_specs=[pl.BlockSpec((1, _C, _H, _W), lambda b: (b, 0, 0, 0))],
        out_specs=pl.BlockSpec((1, 1, 8), lambda b: (b, 0, 0)),
        out_shape=jax.ShapeDtypeStruct((_B, 1, 8), jnp.int32),
    )(output_hm)
    p = p3[:, 0, 0]
    idx0 = index[:, 0].astype(jnp.int32)
    ckp = centerKpoints[:, 0, :].reshape(_B, _K2, 1)
    tgt = target_wh[:, 0, :].reshape(_B, _K2, 1)
    msk = jnp.broadcast_to(mask[:, 0].reshape(_B, 1, 1), (_B, _K2, 1))

    sin3, cos3 = pl.pallas_call(
        _loss_body,
        grid_spec=pltpu.PrefetchScalarGridSpec(
            num_scalar_prefetch=2,
            grid=(_B,),
            in_specs=[
                pl.BlockSpec((1, _K2, 8, 128),
                             lambda b, i0, p_: (b, 0, (i0[b] // _W) // 8,
                                                (i0[b] % _W) // 128)),
                pl.BlockSpec((1, _K2, 8, 128),
                             lambda b, i0, p_: (b, 0, (p_[b] // _W) // 8,
                                                (p_[b] % _W) // 128)),
                pl.BlockSpec((1, _K2, 1), lambda b, i0, p_: (b, 0, 0)),
                pl.BlockSpec((1, _K2, 1), lambda b, i0, p_: (b, 0, 0)),
                pl.BlockSpec((1, _K2, 1), lambda b, i0, p_: (b, 0, 0)),
            ],
            out_specs=[
                pl.BlockSpec((1, 16, 1), lambda b, i0, p_: (b, 0, 0)),
                pl.BlockSpec((1, 16, 1), lambda b, i0, p_: (b, 0, 0)),
            ],
        ),
        out_shape=[jax.ShapeDtypeStruct((_B, 16, 1), jnp.float32)] * 2,
    )(idx0, p, output_wh, output_STA_offset, ckp, tgt, msk)

    sin = sin3[:, 0:12:2, 0].reshape(-1)
    cos = cos3[:, 0:12:2, 0].reshape(-1)
    return sin, cos
